# initial kernel scaffold (unmeasured)
import jax
import jax.numpy as jnp
from jax import lax
from jax.experimental import pallas as pl
from jax.experimental.pallas import tpu as pltpu

N_DEV = 8
SQ = 256
SKV = 4096
H_LOC = 8
DH = 128
D_MODEL = 1024
HD = H_LOC * DH
BLK = 64
SCALE = 0.08838834764831843


def kernel(x, Wq, K_ext, V_ext, Wo):
    pos = lax.axis_index("i")
    x2 = x.reshape(SQ, D_MODEL)
    Wq_s = lax.dynamic_slice_in_dim(Wq, pos * HD, HD, axis=1)
    Wo_s = lax.dynamic_slice_in_dim(Wo, pos * HD, HD, axis=0)
    K2 = K_ext.reshape(SKV, HD)
    V2 = V_ext.reshape(SKV, HD)

    def body(x_ref, wq_ref, k_ref, v_ref, wo_ref, out_ref,
             kbuf, vbuf, comm, ksem, vsem, send_sems, recv_sems):
        my = lax.axis_index("i")
        left = lax.rem(my + (N_DEV - 1), N_DEV)
        right = lax.rem(my + 1, N_DEV)

        barrier = pltpu.get_barrier_semaphore()
        for nbr in (left, right):
            pl.semaphore_signal(
                barrier, inc=1,
                device_id=(nbr,), device_id_type=pl.DeviceIdType.MESH,
            )
        pl.semaphore_wait(barrier, 2)

        def kv_copies(h, slot):
            ck = pltpu.make_async_copy(
                k_ref.at[:, pl.ds(h * DH, DH)], kbuf.at[slot], ksem.at[slot])
            cv = pltpu.make_async_copy(
                v_ref.at[:, pl.ds(h * DH, DH)], vbuf.at[slot], vsem.at[slot])
            return ck, cv

        pending = {0: kv_copies(0, 0)}
        pending[0][0].start()
        pending[0][1].start()

        q = jnp.dot(x_ref[:, :], wq_ref[:, :],
                    preferred_element_type=jnp.float32)

        qb = lax.broadcasted_iota(jnp.int32, (SQ, SKV), 0) // BLK
        kb = lax.broadcasted_iota(jnp.int32, (SQ, SKV), 1) // BLK
        mask = (qb == kb) | (kb == 0) | (((qb + kb) % 3) == 0)

        partial = jnp.zeros((SQ, HD), jnp.float32)
        for h in range(H_LOC):
            slot = h % 2
            ck, cv = pending.pop(h)
            ck.wait()
            cv.wait()
            if h + 1 < H_LOC:
                nxt = kv_copies(h + 1, (h + 1) % 2)
                pending[h + 1] = nxt
                nxt[0].start()
                nxt[1].start()
            q_h = q[:, h * DH:(h + 1) * DH]
            s = lax.dot_general(
                q_h, kbuf[slot],
                (((1,), (1,)), ((), ())),
                preferred_element_type=jnp.float32,
            ) * SCALE
            s = jnp.where(mask, s, -1e9)
            m = jnp.max(s, axis=1, keepdims=True)
            w = jnp.exp(s - m)
            denom = jnp.sum(w, axis=1, keepdims=True)
            ctx = jnp.dot(w, vbuf[slot],
                          preferred_element_type=jnp.float32) / denom
            partial = partial + jnp.dot(
                ctx, wo_ref[h * DH:(h + 1) * DH, :],
                preferred_element_type=jnp.float32)

        comm[0, :, :] = partial
        total = partial
        for hop in range(N_DEV - 1):
            rdma = pltpu.make_async_remote_copy(
                src_ref=comm.at[hop],
                dst_ref=comm.at[hop + 1],
                send_sem=send_sems.at[hop],
                recv_sem=recv_sems.at[hop + 1],
                device_id=(right,),
                device_id_type=pl.DeviceIdType.MESH,
            )
            rdma.start()
            rdma.wait()
            total = total + comm[hop + 1]
        out_ref[0, :, :] = total

    return pl.pallas_call(
        body,
        out_shape=jax.ShapeDtypeStruct((1, SQ, D_MODEL), jnp.float32),
        in_specs=[
            pl.BlockSpec(memory_space=pltpu.VMEM),
            pl.BlockSpec(memory_space=pltpu.VMEM),
            pl.BlockSpec(memory_space=pltpu.ANY),
            pl.BlockSpec(memory_space=pltpu.ANY),
            pl.BlockSpec(memory_space=pltpu.VMEM),
        ],
        out_specs=pl.BlockSpec(memory_space=pltpu.VMEM),
        scratch_shapes=[
            pltpu.VMEM((2, SKV, DH), jnp.float32),
            pltpu.VMEM((2, SKV, DH), jnp.float32),
            pltpu.VMEM((N_DEV, SQ, D_MODEL), jnp.float32),
            pltpu.SemaphoreType.DMA((2,)),
            pltpu.SemaphoreType.DMA((2,)),
            pltpu.SemaphoreType.DMA((N_DEV,)),
            pltpu.SemaphoreType.DMA((N_DEV,)),
        ],
        compiler_params=pltpu.CompilerParams(collective_id=0),
    )(x2, Wq_s, K2, V2, Wo_s)


# baseline (device time: 157730 ns/iter reference)
import jax
import jax.numpy as jnp
from jax import lax
from jax.experimental import pallas as pl
from jax.experimental.pallas import tpu as pltpu

N_DEV = 8
SQ = 256
SKV = 4096
H_LOC = 8
DH = 128
D_MODEL = 1024
HD = H_LOC * DH
BLK = 64
SCALE = 0.08838834764831843


def kernel(x, Wq, K_ext, V_ext, Wo):
    pos = lax.axis_index("i")
    x2 = x.reshape(SQ, D_MODEL)
    Wq_s = lax.dynamic_slice_in_dim(Wq, pos * HD, HD, axis=1)
    Wo_s = lax.dynamic_slice_in_dim(Wo, pos * HD, HD, axis=0)
    K2 = K_ext.reshape(SKV, HD)
    V2 = V_ext.reshape(SKV, HD)

    def body(x_ref, wq_ref, k_ref, v_ref, wo_ref, out_ref,
             kbuf, vbuf, comm, ksem, vsem, send_sems, recv_sems):
        my = lax.axis_index("i")
        left = lax.rem(my + (N_DEV - 1), N_DEV)
        right = lax.rem(my + 1, N_DEV)

        barrier = pltpu.get_barrier_semaphore()
        for nbr in (left, right):
            pl.semaphore_signal(
                barrier, inc=1,
                device_id=(nbr,), device_id_type=pl.DeviceIdType.MESH,
            )
        pl.semaphore_wait(barrier, 2)

        def kv_copies(h, slot):
            ck = pltpu.make_async_copy(
                k_ref.at[:, pl.ds(h * DH, DH)], kbuf.at[slot], ksem.at[slot])
            cv = pltpu.make_async_copy(
                v_ref.at[:, pl.ds(h * DH, DH)], vbuf.at[slot], vsem.at[slot])
            return ck, cv

        pending = {0: kv_copies(0, 0)}
        pending[0][0].start()
        pending[0][1].start()

        q = jnp.dot(x_ref[:, :], wq_ref[:, :],
                    preferred_element_type=jnp.float32)

        qb = lax.broadcasted_iota(jnp.int32, (SQ, SKV), 0) // BLK
        kb = lax.broadcasted_iota(jnp.int32, (SQ, SKV), 1) // BLK
        mask = (qb == kb) | (kb == 0) | (((qb + kb) % 3) == 0)

        partial = jnp.zeros((SQ, HD), jnp.float32)
        for h in range(H_LOC):
            slot = h % 2
            ck, cv = pending.pop(h)
            ck.wait()
            cv.wait()
            if h + 1 < H_LOC:
                nxt = kv_copies(h + 1, (h + 1) % 2)
                pending[h + 1] = nxt
                nxt[0].start()
                nxt[1].start()
            q_h = q[:, h * DH:(h + 1) * DH]
            s = lax.dot_general(
                q_h, kbuf[slot],
                (((1,), (1,)), ((), ())),
                preferred_element_type=jnp.float32,
            ) * SCALE
            s = jnp.where(mask, s, -1e9)
            m = jnp.max(s, axis=1, keepdims=True)
            w = jnp.exp(s - m)
            denom = jnp.sum(w, axis=1, keepdims=True)
            ctx = jnp.dot(w, vbuf[slot],
                          preferred_element_type=jnp.float32) / denom
            partial = partial + jnp.dot(
                ctx, wo_ref[h * DH:(h + 1) * DH, :],
                preferred_element_type=jnp.float32)

        comm[0, :, :] = partial
        total = partial
        for hop in range(N_DEV - 1):
            rdma = pltpu.make_async_remote_copy(
                src_ref=comm.at[hop],
                dst_ref=comm.at[hop + 1],
                send_sem=send_sems.at[hop],
                recv_sem=recv_sems.at[hop + 1],
                device_id=(right,),
                device_id_type=pl.DeviceIdType.MESH,
            )
            rdma.start()
            rdma.wait()
            total = total + comm[hop + 1]
        out_ref[0, :, :] = total

    return pl.pallas_call(
        body,
        out_shape=jax.ShapeDtypeStruct((1, SQ, D_MODEL), jnp.float32),
        in_specs=[
            pl.BlockSpec(memory_space=pltpu.VMEM),
            pl.BlockSpec(memory_space=pltpu.VMEM),
            pl.BlockSpec(memory_space=pl.ANY),
            pl.BlockSpec(memory_space=pl.ANY),
            pl.BlockSpec(memory_space=pltpu.VMEM),
        ],
        out_specs=pl.BlockSpec(memory_space=pltpu.VMEM),
        scratch_shapes=[
            pltpu.VMEM((2, SKV, DH), jnp.float32),
            pltpu.VMEM((2, SKV, DH), jnp.float32),
            pltpu.VMEM((N_DEV, SQ, D_MODEL), jnp.float32),
            pltpu.SemaphoreType.DMA((2,)),
            pltpu.SemaphoreType.DMA((2,)),
            pltpu.SemaphoreType.DMA((N_DEV,)),
            pltpu.SemaphoreType.DMA((N_DEV,)),
        ],
        compiler_params=pltpu.CompilerParams(collective_id=0),
    )(x2, Wq_s, K2, V2, Wo_s)


# device time: 95383 ns/iter; 1.6536x vs baseline; 1.6536x over previous
import jax
import jax.numpy as jnp
from jax import lax
from jax.experimental import pallas as pl
from jax.experimental.pallas import tpu as pltpu

N_DEV = 8
SQ = 256
SKV = 4096
H_LOC = 8
DH = 128
D_MODEL = 1024
HD = H_LOC * DH
BLK = 64
SCALE = 0.08838834764831843


def kernel(x, Wq, K_ext, V_ext, Wo):
    pos = lax.axis_index("i")
    x2 = x.reshape(SQ, D_MODEL)
    Wq_s = lax.dynamic_slice_in_dim(Wq, pos * HD, HD, axis=1)
    Wo_s = lax.dynamic_slice_in_dim(Wo, pos * HD, HD, axis=0)
    K2 = K_ext.reshape(SKV, HD)
    V2 = V_ext.reshape(SKV, HD)

    def body(x_ref, wq_ref, k_ref, v_ref, wo_ref, out_ref,
             kbuf, vbuf, wbuf, rs0, rs1, rs2,
             ksem, vsem, rs_send, rs_recv, ag_send, ag_recv):
        my = lax.axis_index("i")
        v = my ^ ((my >> 1) & 1)
        partners = [(v ^ (1 << r)) ^ (((v ^ (1 << r)) >> 1) & 1)
                    for r in range(3)]

        barrier = pltpu.get_barrier_semaphore()
        for nbr in partners:
            pl.semaphore_signal(
                barrier, inc=1,
                device_id=(nbr,), device_id_type=pl.DeviceIdType.MESH,
            )
        pl.semaphore_wait(barrier, 3)

        def kv_copies(h, slot):
            ck = pltpu.make_async_copy(
                k_ref.at[:, pl.ds(h * DH, DH)], kbuf.at[slot], ksem.at[slot])
            cv = pltpu.make_async_copy(
                v_ref.at[:, pl.ds(h * DH, DH)], vbuf.at[slot], vsem.at[slot])
            return ck, cv

        pending = {0: kv_copies(0, 0)}
        pending[0][0].start()
        pending[0][1].start()

        q = jnp.dot(x_ref[:, :], wq_ref[:, :],
                    preferred_element_type=jnp.float32)

        qb = lax.broadcasted_iota(jnp.int32, (SQ, SKV), 0) // BLK
        kb = lax.broadcasted_iota(jnp.int32, (SQ, SKV), 1) // BLK
        mask = (qb == kb) | (kb == 0) | (((qb + kb) % 3) == 0)

        partial = jnp.zeros((SQ, HD), jnp.float32)
        for h in range(H_LOC):
            slot = h % 2
            ck, cv = pending.pop(h)
            ck.wait()
            cv.wait()
            if h + 1 < H_LOC:
                nxt = kv_copies(h + 1, (h + 1) % 2)
                pending[h + 1] = nxt
                nxt[0].start()
                nxt[1].start()
            q_h = q[:, h * DH:(h + 1) * DH]
            s = lax.dot_general(
                q_h, kbuf[slot],
                (((1,), (1,)), ((), ())),
                preferred_element_type=jnp.float32,
            ) * SCALE
            s = jnp.where(mask, s, -1e9)
            m = jnp.max(s, axis=1, keepdims=True)
            w = jnp.exp(s - m)
            denom = jnp.sum(w, axis=1, keepdims=True)
            ctx = jnp.dot(w, vbuf[slot],
                          preferred_element_type=jnp.float32) / denom
            partial = partial + jnp.dot(
                ctx, wo_ref[h * DH:(h + 1) * DH, :],
                preferred_element_type=jnp.float32)

        wbuf[:, :] = partial

        rsbufs = (rs0, rs1, rs2)
        base = my * 0
        for r, half in enumerate((SQ // 2, SQ // 4, SQ // 8)):
            bit = (v >> r) & 1
            send_off = base + (1 - bit) * half
            keep_off = base + bit * half
            rdma = pltpu.make_async_remote_copy(
                src_ref=wbuf.at[pl.ds(send_off, half), :],
                dst_ref=rsbufs[r],
                send_sem=rs_send.at[r],
                recv_sem=rs_recv.at[r],
                device_id=(partners[r],),
                device_id_type=pl.DeviceIdType.MESH,
            )
            rdma.start()
            rdma.wait()
            wbuf[pl.ds(keep_off, half), :] = (
                wbuf[pl.ds(keep_off, half), :] + rsbufs[r][:, :])
            base = keep_off

        for idx, (r, blk) in enumerate(((2, SQ // 8), (1, SQ // 4), (0, SQ // 2))):
            bit = (v >> r) & 1
            rdma = pltpu.make_async_remote_copy(
                src_ref=wbuf.at[pl.ds(base, blk), :],
                dst_ref=wbuf.at[pl.ds(base, blk), :],
                send_sem=ag_send.at[idx],
                recv_sem=ag_recv.at[idx],
                device_id=(partners[r],),
                device_id_type=pl.DeviceIdType.MESH,
            )
            rdma.start()
            rdma.wait()
            base = base - bit * blk

        out_ref[0, :, :] = wbuf[:, :]

    return pl.pallas_call(
        body,
        out_shape=jax.ShapeDtypeStruct((1, SQ, D_MODEL), jnp.float32),
        in_specs=[
            pl.BlockSpec(memory_space=pltpu.VMEM),
            pl.BlockSpec(memory_space=pltpu.VMEM),
            pl.BlockSpec(memory_space=pl.ANY),
            pl.BlockSpec(memory_space=pl.ANY),
            pl.BlockSpec(memory_space=pltpu.VMEM),
        ],
        out_specs=pl.BlockSpec(memory_space=pltpu.VMEM),
        scratch_shapes=[
            pltpu.VMEM((2, SKV, DH), jnp.float32),
            pltpu.VMEM((2, SKV, DH), jnp.float32),
            pltpu.VMEM((SQ, D_MODEL), jnp.float32),
            pltpu.VMEM((SQ // 2, D_MODEL), jnp.float32),
            pltpu.VMEM((SQ // 4, D_MODEL), jnp.float32),
            pltpu.VMEM((SQ // 8, D_MODEL), jnp.float32),
            pltpu.SemaphoreType.DMA((2,)),
            pltpu.SemaphoreType.DMA((2,)),
            pltpu.SemaphoreType.DMA((3,)),
            pltpu.SemaphoreType.DMA((3,)),
            pltpu.SemaphoreType.DMA((3,)),
            pltpu.SemaphoreType.DMA((3,)),
        ],
        compiler_params=pltpu.CompilerParams(collective_id=0),
    )(x2, Wq_s, K2, V2, Wo_s)


# device time: 84904 ns/iter; 1.8577x vs baseline; 1.1234x over previous
import jax
import jax.numpy as jnp
from jax import lax
from jax.experimental import pallas as pl
from jax.experimental.pallas import tpu as pltpu

N_DEV = 8
SQ = 256
SKV = 4096
H_LOC = 8
DH = 128
D_MODEL = 1024
HD = H_LOC * DH
BLK = 64
SCALE = 0.08838834764831843


def kernel(x, Wq, K_ext, V_ext, Wo):

    def body(x_ref, wq_ref, k_ref, v_ref, wo_ref, out_ref,
             wq_v, wo_v, kbuf, vbuf, wbuf, rs0, rs1, rs2,
             wq_sem, wo_sem, ksem, vsem, rs_send, rs_recv, ag_send, ag_recv):
        my = lax.axis_index("i")
        v = my ^ ((my >> 1) & 1)
        partners = [(v ^ (1 << r)) ^ (((v ^ (1 << r)) >> 1) & 1)
                    for r in range(3)]

        cwq = pltpu.make_async_copy(
            wq_ref.at[:, pl.ds(my * HD, HD)], wq_v, wq_sem)
        cwo = pltpu.make_async_copy(
            wo_ref.at[pl.ds(my * HD, HD), :], wo_v, wo_sem)
        cwq.start()
        cwo.start()

        def kv_copies(h, slot):
            ck = pltpu.make_async_copy(
                k_ref.at[0, :, h, :], kbuf.at[slot], ksem.at[slot])
            cv = pltpu.make_async_copy(
                v_ref.at[0, :, h, :], vbuf.at[slot], vsem.at[slot])
            return ck, cv

        pending = {0: kv_copies(0, 0)}
        pending[0][0].start()
        pending[0][1].start()

        barrier = pltpu.get_barrier_semaphore()
        for nbr in partners:
            pl.semaphore_signal(
                barrier, inc=1,
                device_id=(nbr,), device_id_type=pl.DeviceIdType.MESH,
            )
        pl.semaphore_wait(barrier, 3)

        qb = lax.broadcasted_iota(jnp.int32, (SQ, SKV), 0) // BLK
        kb = lax.broadcasted_iota(jnp.int32, (SQ, SKV), 1) // BLK
        mask = (qb == kb) | (kb == 0) | (((qb + kb) % 3) == 0)

        cwq.wait()
        q = jnp.dot(x_ref[0], wq_v[:, :], preferred_element_type=jnp.float32)

        partial = jnp.zeros((SQ, HD), jnp.float32)
        for h in range(H_LOC):
            slot = h % 2
            ck, cv = pending.pop(h)
            ck.wait()
            cv.wait()
            if h + 1 < H_LOC:
                nxt = kv_copies(h + 1, (h + 1) % 2)
                pending[h + 1] = nxt
                nxt[0].start()
                nxt[1].start()
            if h == 0:
                cwo.wait()
            q_h = q[:, h * DH:(h + 1) * DH]
            s = lax.dot_general(
                q_h, kbuf[slot],
                (((1,), (1,)), ((), ())),
                preferred_element_type=jnp.float32,
            ) * SCALE
            s = jnp.where(mask, s, -1e9)
            m = jnp.max(s, axis=1, keepdims=True)
            w = jnp.exp(s - m)
            denom = jnp.sum(w, axis=1, keepdims=True)
            ctx = jnp.dot(w, vbuf[slot],
                          preferred_element_type=jnp.float32) / denom
            partial = partial + jnp.dot(
                ctx, wo_v[h * DH:(h + 1) * DH, :],
                preferred_element_type=jnp.float32)

        wbuf[:, :] = partial

        rsbufs = (rs0, rs1, rs2)
        base = my * 0
        for r, half in enumerate((SQ // 2, SQ // 4, SQ // 8)):
            bit = (v >> r) & 1
            send_off = base + (1 - bit) * half
            keep_off = base + bit * half
            rdma = pltpu.make_async_remote_copy(
                src_ref=wbuf.at[pl.ds(send_off, half), :],
                dst_ref=rsbufs[r],
                send_sem=rs_send.at[r],
                recv_sem=rs_recv.at[r],
                device_id=(partners[r],),
                device_id_type=pl.DeviceIdType.MESH,
            )
            rdma.start()
            rdma.wait()
            wbuf[pl.ds(keep_off, half), :] = (
                wbuf[pl.ds(keep_off, half), :] + rsbufs[r][:, :])
            base = keep_off

        for idx, (r, blk) in enumerate(((2, SQ // 8), (1, SQ // 4), (0, SQ // 2))):
            bit = (v >> r) & 1
            rdma = pltpu.make_async_remote_copy(
                src_ref=wbuf.at[pl.ds(base, blk), :],
                dst_ref=wbuf.at[pl.ds(base, blk), :],
                send_sem=ag_send.at[idx],
                recv_sem=ag_recv.at[idx],
                device_id=(partners[r],),
                device_id_type=pl.DeviceIdType.MESH,
            )
            rdma.start()
            rdma.wait()
            base = base - bit * blk

        out_ref[0, :, :] = wbuf[:, :]

    return pl.pallas_call(
        body,
        out_shape=jax.ShapeDtypeStruct((1, SQ, D_MODEL), jnp.float32),
        in_specs=[
            pl.BlockSpec(memory_space=pltpu.VMEM),
            pl.BlockSpec(memory_space=pl.ANY),
            pl.BlockSpec(memory_space=pl.ANY),
            pl.BlockSpec(memory_space=pl.ANY),
            pl.BlockSpec(memory_space=pl.ANY),
        ],
        out_specs=pl.BlockSpec(memory_space=pltpu.VMEM),
        scratch_shapes=[
            pltpu.VMEM((D_MODEL, HD), jnp.float32),
            pltpu.VMEM((HD, D_MODEL), jnp.float32),
            pltpu.VMEM((2, SKV, DH), jnp.float32),
            pltpu.VMEM((2, SKV, DH), jnp.float32),
            pltpu.VMEM((SQ, D_MODEL), jnp.float32),
            pltpu.VMEM((SQ // 2, D_MODEL), jnp.float32),
            pltpu.VMEM((SQ // 4, D_MODEL), jnp.float32),
            pltpu.VMEM((SQ // 8, D_MODEL), jnp.float32),
            pltpu.SemaphoreType.DMA,
            pltpu.SemaphoreType.DMA,
            pltpu.SemaphoreType.DMA((2,)),
            pltpu.SemaphoreType.DMA((2,)),
            pltpu.SemaphoreType.DMA((3,)),
            pltpu.SemaphoreType.DMA((3,)),
            pltpu.SemaphoreType.DMA((3,)),
            pltpu.SemaphoreType.DMA((3,)),
        ],
        compiler_params=pltpu.CompilerParams(collective_id=0),
    )(x, Wq, K_ext, V_ext, Wo)


# device time: 77976 ns/iter; 2.0228x vs baseline; 1.0888x over previous
import jax
import jax.numpy as jnp
from jax import lax
from jax.experimental import pallas as pl
from jax.experimental.pallas import tpu as pltpu

N_DEV = 8
SQ = 256
SKV = 4096
H_LOC = 8
DH = 128
D_MODEL = 1024
HD = H_LOC * DH
BLK = 64
SCALE = 0.08838834764831843


def kernel(x, Wq, K_ext, V_ext, Wo):

    def body(x_ref, wq_ref, k_ref, v_ref, wo_ref, out_ref,
             wq_v, wo_v, kbuf, vbuf, wbuf, rs0, rs1, rs2,
             wq_sem, wo_sem, ksem, vsem, rs_send, rs_recv, ag_send, ag_recv):
        my = lax.axis_index("i")
        v = my ^ ((my >> 1) & 1)
        partners = [(v ^ (1 << r)) ^ (((v ^ (1 << r)) >> 1) & 1)
                    for r in range(3)]

        cwq = pltpu.make_async_copy(
            wq_ref.at[:, pl.ds(my * HD, HD)], wq_v, wq_sem)
        cwo = pltpu.make_async_copy(
            wo_ref.at[pl.ds(my * HD, HD), :], wo_v, wo_sem)
        cwq.start()
        cwo.start()

        def kv_copies(h):
            ck = pltpu.make_async_copy(
                k_ref.at[0, :, h, :], kbuf.at[h], ksem.at[h])
            cv = pltpu.make_async_copy(
                v_ref.at[0, :, h, :], vbuf.at[h], vsem.at[h])
            return ck, cv

        pending = {}
        for h in range(H_LOC):
            pending[h] = kv_copies(h)
            pending[h][0].start()
            pending[h][1].start()

        barrier = pltpu.get_barrier_semaphore()
        for nbr in partners:
            pl.semaphore_signal(
                barrier, inc=1,
                device_id=(nbr,), device_id_type=pl.DeviceIdType.MESH,
            )
        pl.semaphore_wait(barrier, 3)

        qb = lax.broadcasted_iota(jnp.int32, (SQ, SKV), 0) // BLK
        kb = lax.broadcasted_iota(jnp.int32, (SQ, SKV), 1) // BLK
        mask = (qb == kb) | (kb == 0) | (((qb + kb) % 3) == 0)

        cwq.wait()
        q = jnp.dot(x_ref[0], wq_v[:, :], preferred_element_type=jnp.float32)

        partial = jnp.zeros((SQ, HD), jnp.float32)
        for h in range(H_LOC):
            ck, cv = pending.pop(h)
            ck.wait()
            cv.wait()
            if h == 0:
                cwo.wait()
            q_h = q[:, h * DH:(h + 1) * DH]
            s = lax.dot_general(
                q_h, kbuf[h],
                (((1,), (1,)), ((), ())),
                preferred_element_type=jnp.float32,
            ) * SCALE
            s = jnp.where(mask, s, -1e9)
            m = jnp.max(s, axis=1, keepdims=True)
            w = jnp.exp(s - m)
            denom = jnp.sum(w, axis=1, keepdims=True)
            ctx = jnp.dot(w, vbuf[h],
                          preferred_element_type=jnp.float32) / denom
            partial = partial + jnp.dot(
                ctx, wo_v[h * DH:(h + 1) * DH, :],
                preferred_element_type=jnp.float32)

        wbuf[:, :] = partial

        rsbufs = (rs0, rs1, rs2)
        base = my * 0
        for r, half in enumerate((SQ // 2, SQ // 4, SQ // 8)):
            bit = (v >> r) & 1
            send_off = base + (1 - bit) * half
            keep_off = base + bit * half
            rdma = pltpu.make_async_remote_copy(
                src_ref=wbuf.at[pl.ds(send_off, half), :],
                dst_ref=rsbufs[r],
                send_sem=rs_send.at[r],
                recv_sem=rs_recv.at[r],
                device_id=(partners[r],),
                device_id_type=pl.DeviceIdType.MESH,
            )
            rdma.start()
            rdma.wait()
            wbuf[pl.ds(keep_off, half), :] = (
                wbuf[pl.ds(keep_off, half), :] + rsbufs[r][:, :])
            base = keep_off

        for idx, (r, blk) in enumerate(((2, SQ // 8), (1, SQ // 4), (0, SQ // 2))):
            bit = (v >> r) & 1
            rdma = pltpu.make_async_remote_copy(
                src_ref=wbuf.at[pl.ds(base, blk), :],
                dst_ref=wbuf.at[pl.ds(base, blk), :],
                send_sem=ag_send.at[idx],
                recv_sem=ag_recv.at[idx],
                device_id=(partners[r],),
                device_id_type=pl.DeviceIdType.MESH,
            )
            rdma.start()
            rdma.wait()
            base = base - bit * blk

        out_ref[0, :, :] = wbuf[:, :]

    return pl.pallas_call(
        body,
        out_shape=jax.ShapeDtypeStruct((1, SQ, D_MODEL), jnp.float32),
        in_specs=[
            pl.BlockSpec(memory_space=pltpu.VMEM),
            pl.BlockSpec(memory_space=pl.ANY),
            pl.BlockSpec(memory_space=pl.ANY),
            pl.BlockSpec(memory_space=pl.ANY),
            pl.BlockSpec(memory_space=pl.ANY),
        ],
        out_specs=pl.BlockSpec(memory_space=pltpu.VMEM),
        scratch_shapes=[
            pltpu.VMEM((D_MODEL, HD), jnp.float32),
            pltpu.VMEM((HD, D_MODEL), jnp.float32),
            pltpu.VMEM((H_LOC, SKV, DH), jnp.float32),
            pltpu.VMEM((H_LOC, SKV, DH), jnp.float32),
            pltpu.VMEM((SQ, D_MODEL), jnp.float32),
            pltpu.VMEM((SQ // 2, D_MODEL), jnp.float32),
            pltpu.VMEM((SQ // 4, D_MODEL), jnp.float32),
            pltpu.VMEM((SQ // 8, D_MODEL), jnp.float32),
            pltpu.SemaphoreType.DMA,
            pltpu.SemaphoreType.DMA,
            pltpu.SemaphoreType.DMA((H_LOC,)),
            pltpu.SemaphoreType.DMA((H_LOC,)),
            pltpu.SemaphoreType.DMA((3,)),
            pltpu.SemaphoreType.DMA((3,)),
            pltpu.SemaphoreType.DMA((3,)),
            pltpu.SemaphoreType.DMA((3,)),
        ],
        compiler_params=pltpu.CompilerParams(
            collective_id=0,
            vmem_limit_bytes=64 * 1024 * 1024,
        ),
    )(x, Wq, K_ext, V_ext, Wo)
